# baseline (speedup 1.0000x reference)
import functools

import jax
import jax.numpy as jnp
from jax import lax
from jax.experimental import pallas as pl
from jax.experimental.pallas import tpu as pltpu
from jax.experimental.pallas import tpu_sc as plsc

N = 10000
E = 320000
D = 128
DH = D // 2
OUT = 128

NC = 2
NS = 16

N_PAD = 10240
ROWS_PER_TILE = N_PAD // NS
E_PAD = 327680
EDGES_PER_TILE = E_PAD // NS
C = 256
NCHUNK = EDGES_PER_TILE // C
CB = C // 128
H_CHUNKS = 30
ROWS2D = EDGES_PER_TILE // 128


def _hdelta_body(hs_ref, hb_ref, out_ref):
    hd = hs_ref[...] - hb_ref[...]
    out_ref[0, pl.ds(0, N)] = hd[:, :DH]
    out_ref[1, pl.ds(0, N)] = hd[:, DH:]


def _hdelta(H_src, HBar_src):
    return pl.pallas_call(
        _hdelta_body,
        out_shape=jax.ShapeDtypeStruct((2, N_PAD, DH), jnp.float32),
    )(H_src, HBar_src)


def _sc_body(hd2_hbm, src1d_hbm, dst2d_hbm, zrows_hbm, zdeg_hbm, ones_hbm,
             sumL_hbm, sumR_hbm, deg0_hbm, deg1_hbm,
             idxs_v, idxd_v, rows_v, ones_v, deg_v, table_sh, accum_sh, deg_sh,
             sem_i0, sem_i1, sem_g0, sem_g1, sem_s0, sem_s1):
    cid = lax.axis_index("c")
    sid = lax.axis_index("s")
    sem_i = (sem_i0, sem_i1)
    sem_g = (sem_g0, sem_g1)
    sem_s = (sem_s0, sem_s1)
    rbase = sid * ROWS_PER_TILE

    pltpu.sync_copy(zrows_hbm, rows_v.at[0, pl.ds(0, 128)])
    for j in range(ROWS_PER_TILE // 128):
        pltpu.sync_copy(rows_v.at[0, pl.ds(0, 128)],
                        accum_sh.at[pl.ds(rbase + j * 128, 128)])
    pltpu.sync_copy(zdeg_hbm, deg_v)
    pltpu.sync_copy(deg_v, deg_sh.at[pl.ds(rbase, ROWS_PER_TILE)])
    pltpu.sync_copy(ones_hbm, ones_v)

    table_hbm = hd2_hbm.at[cid]
    for j in range(2):
        pltpu.sync_copy(table_hbm.at[pl.ds(rbase + j * 256, 256)],
                        rows_v.at[0])
        pltpu.sync_copy(rows_v.at[0],
                        table_sh.at[pl.ds(rbase + j * 256, 256)])
    pltpu.sync_copy(table_hbm.at[pl.ds(rbase + 512, 128)],
                    rows_v.at[1, pl.ds(0, 128)])
    pltpu.sync_copy(rows_v.at[1, pl.ds(0, 128)],
                    table_sh.at[pl.ds(rbase + 512, 128)])
    plsc.subcore_barrier()

    row_base = sid * ROWS2D

    def start_idx(g, b):
        e0 = (row_base + g * CB) * 128
        r0 = row_base + g * CB
        pltpu.async_copy(src1d_hbm.at[pl.ds(e0, C)], idxs_v.at[b], sem_i[b])
        pltpu.async_copy(dst2d_hbm.at[pl.ds(r0, CB)], idxd_v.at[b], sem_i[b])

    def wait_idx(g, b):
        e0 = (row_base + g * CB) * 128
        r0 = row_base + g * CB
        pltpu.make_async_copy(src1d_hbm.at[pl.ds(e0, C)], idxs_v.at[b],
                              sem_i[b]).wait()
        pltpu.make_async_copy(dst2d_hbm.at[pl.ds(r0, CB)], idxd_v.at[b],
                              sem_i[b]).wait()

    def start_gathers(g, b):
        @pl.when(g < H_CHUNKS)
        def _():
            pltpu.async_copy(table_hbm.at[idxs_v.at[b]], rows_v.at[b],
                             sem_g[b])

        @pl.when(g >= H_CHUNKS)
        def _():
            pltpu.async_copy(table_sh.at[idxs_v.at[b]], rows_v.at[b],
                             sem_g[b])

    def wait_gathers(g, b):
        pltpu.make_async_copy(table_sh.at[idxs_v.at[b]], rows_v.at[b],
                              sem_g[b]).wait()

    def deg_cond(g):
        return (g < NCHUNK // 2) == (cid == 0)

    def start_scatters(g, b):
        for j in range(CB):
            pltpu.async_copy(rows_v.at[b, pl.ds(j * 128, 128)],
                             accum_sh.at[idxd_v.at[b, j]],
                             sem_s[b], add=True)

        @pl.when(deg_cond(g))
        def _():
            for j in range(CB):
                pltpu.async_copy(ones_v, deg_sh.at[idxd_v.at[b, j]],
                                 sem_s[b], add=True)

    def wait_scatters(g, b):
        for j in range(CB):
            pltpu.make_async_copy(rows_v.at[b, pl.ds(j * 128, 128)],
                                  accum_sh.at[idxd_v.at[b, j]],
                                  sem_s[b]).wait()

        @pl.when(deg_cond(g))
        def _():
            for j in range(CB):
                pltpu.make_async_copy(ones_v, deg_sh.at[idxd_v.at[b, j]],
                                      sem_s[b]).wait()

    for b in (0, 1):
        start_idx(b, b)
    for b in (0, 1):
        wait_idx(b, b)
        start_gathers(b, b)

    @pl.loop(0, NCHUNK - 2, step=2)
    def _pair(i):
        for b in (0, 1):
            g = i + b
            wait_gathers(g, b)
            start_scatters(g, b)
        for b in (0, 1):
            g = i + b
            wait_scatters(g, b)
            start_idx(g + 2, b)
            wait_idx(g + 2, b)
            start_gathers(g + 2, b)

    for b in (0, 1):
        g = NCHUNK - 2 + b
        wait_gathers(g, b)
        start_scatters(g, b)
    for b in (0, 1):
        g = NCHUNK - 2 + b
        wait_scatters(g, b)

    plsc.subcore_barrier()

    @pl.when(cid == 0)
    def _():
        for j in range(5):
            pltpu.sync_copy(accum_sh.at[pl.ds(rbase + j * 128, 128)],
                            rows_v.at[0, pl.ds(0, 128)])
            pltpu.sync_copy(rows_v.at[0, pl.ds(0, 128)],
                            sumL_hbm.at[pl.ds(rbase + j * 128, 128)])
        pltpu.sync_copy(deg_sh.at[pl.ds(rbase, ROWS_PER_TILE)], deg_v)
        pltpu.sync_copy(deg_v, deg0_hbm.at[pl.ds(rbase, ROWS_PER_TILE)])

    @pl.when(cid == 1)
    def _():
        for j in range(5):
            pltpu.sync_copy(accum_sh.at[pl.ds(rbase + j * 128, 128)],
                            rows_v.at[0, pl.ds(0, 128)])
            pltpu.sync_copy(rows_v.at[0, pl.ds(0, 128)],
                            sumR_hbm.at[pl.ds(rbase + j * 128, 128)])
        pltpu.sync_copy(deg_sh.at[pl.ds(rbase, ROWS_PER_TILE)], deg_v)
        pltpu.sync_copy(deg_v, deg1_hbm.at[pl.ds(rbase, ROWS_PER_TILE)])


def _sc_scatter(hd2, src1d, dst2d, zrows, zdeg, ones128):
    mesh = plsc.VectorSubcoreMesh(core_axis_name="c", subcore_axis_name="s")
    f = functools.partial(
        pl.kernel,
        out_type=[
            jax.ShapeDtypeStruct((N_PAD, DH), jnp.float32),
            jax.ShapeDtypeStruct((N_PAD, DH), jnp.float32),
            jax.ShapeDtypeStruct((N_PAD,), jnp.float32),
            jax.ShapeDtypeStruct((N_PAD,), jnp.float32),
        ],
        mesh=mesh,
        scratch_types=[
            pltpu.VMEM((2, C), jnp.int32),
            pltpu.VMEM((2, CB, 128), jnp.int32),
            pltpu.VMEM((2, C, DH), jnp.float32),
            pltpu.VMEM((128,), jnp.float32),
            pltpu.VMEM((ROWS_PER_TILE,), jnp.float32),
            pltpu.VMEM_SHARED((N_PAD, DH), jnp.float32),
            pltpu.VMEM_SHARED((N_PAD, DH), jnp.float32),
            pltpu.VMEM_SHARED((N_PAD,), jnp.float32),
            pltpu.SemaphoreType.DMA,
            pltpu.SemaphoreType.DMA,
            pltpu.SemaphoreType.DMA,
            pltpu.SemaphoreType.DMA,
            pltpu.SemaphoreType.DMA,
            pltpu.SemaphoreType.DMA,
        ],
        compiler_params=pltpu.CompilerParams(use_tc_tiling_on_sc=False),
    )(_sc_body)
    return f(hd2, src1d, dst2d, zrows, zdeg, ones128)


def _combine_body(sL_ref, sR_ref, d0_ref, d1_ref, hd_ref, agg_ref, w_ref, b_ref,
                  out_ref):
    deg = jnp.maximum(d0_ref[...] + d1_ref[...], 1.0)
    inv = (1.0 / deg)[:, None]
    agg = agg_ref[...]
    hnL = agg[:, :DH] + sL_ref[...] * inv
    hnR = agg[:, DH:] + sR_ref[...] * inv
    acc = jnp.dot(hd_ref[...], w_ref[0:D, :], preferred_element_type=jnp.float32)
    acc = acc + jnp.dot(hnL, w_ref[D:D + DH, :], preferred_element_type=jnp.float32)
    acc = acc + jnp.dot(hnR, w_ref[D + DH:2 * D, :], preferred_element_type=jnp.float32)
    out_ref[...] = acc + b_ref[...][None, :]


def _combine(sL, sR, d0, d1, H_dst, agg, W, b):
    R = 1024
    grid = (N_PAD // R,)
    return pl.pallas_call(
        _combine_body,
        grid=grid,
        in_specs=[
            pl.BlockSpec((R, DH), lambda i: (i, 0)),
            pl.BlockSpec((R, DH), lambda i: (i, 0)),
            pl.BlockSpec((R,), lambda i: (i,)),
            pl.BlockSpec((R,), lambda i: (i,)),
            pl.BlockSpec((R, D), lambda i: (i, 0)),
            pl.BlockSpec((R, D), lambda i: (i, 0)),
            pl.BlockSpec((2 * D, OUT), lambda i: (0, 0)),
            pl.BlockSpec((OUT,), lambda i: (0,)),
        ],
        out_specs=pl.BlockSpec((R, OUT), lambda i: (i, 0)),
        out_shape=jax.ShapeDtypeStruct((N, OUT), jnp.float32),
    )(sL, sR, d0, d1, H_dst, agg, W, b)


def kernel(H_src, H_dst, HBar_src, agg_HBar_dst, edge_index, W, b):
    hd2 = _hdelta(H_src, HBar_src)

    src = edge_index[0]
    dst = edge_index[1]
    pad = E_PAD - E
    src_pad = jnp.concatenate([src, jnp.zeros((pad,), jnp.int32)])
    dst_pad = jnp.concatenate([dst, jnp.full((pad,), N, jnp.int32)])
    dst2d = dst_pad.reshape(E_PAD // 128, 128)

    zrows = jnp.zeros((128, DH), jnp.float32)
    zdeg = jnp.zeros((ROWS_PER_TILE,), jnp.float32)
    ones128 = jnp.ones((128,), jnp.float32)

    sL, sR, d0, d1 = _sc_scatter(hd2, src_pad, dst2d, zrows, zdeg, ones128)

    return _combine(sL, sR, d0, d1, H_dst, agg_HBar_dst, W, b)


# speedup vs baseline: 1.0401x; 1.0401x over previous
import functools

import jax
import jax.numpy as jnp
from jax import lax
from jax.experimental import pallas as pl
from jax.experimental.pallas import tpu as pltpu
from jax.experimental.pallas import tpu_sc as plsc

N = 10000
E = 320000
D = 128
DH = D // 2
OUT = 128

NC = 2
NS = 16

N_PAD = 10240
ROWS_PER_TILE = N_PAD // NS
E_PAD = 327680
EDGES_PER_TILE = E_PAD // NS
C = 256
NCHUNK = EDGES_PER_TILE // C
CB = C // 128
H_CHUNKS = 46
ROWS2D = EDGES_PER_TILE // 128


def _hdelta_body(hs_ref, hb_ref, out_ref):
    hd = hs_ref[...] - hb_ref[...]
    out_ref[0, pl.ds(0, N)] = hd[:, :DH]
    out_ref[1, pl.ds(0, N)] = hd[:, DH:]


def _hdelta(H_src, HBar_src):
    return pl.pallas_call(
        _hdelta_body,
        out_shape=jax.ShapeDtypeStruct((2, N_PAD, DH), jnp.float32),
    )(H_src, HBar_src)


def _sc_body(hd2_hbm, src1d_hbm, dst2d_hbm, zrows_hbm, zdeg_hbm, ones_hbm,
             sumL_hbm, sumR_hbm, deg0_hbm, deg1_hbm,
             idxs_v, idxd_v, rows_v, ones_v, deg_v, table_sh, accum_sh, deg_sh,
             sem_i0, sem_i1, sem_g0, sem_g1, sem_s0, sem_s1):
    cid = lax.axis_index("c")
    sid = lax.axis_index("s")
    sem_i = (sem_i0, sem_i1)
    sem_g = (sem_g0, sem_g1)
    sem_s = (sem_s0, sem_s1)
    rbase = sid * ROWS_PER_TILE

    pltpu.sync_copy(zrows_hbm, rows_v.at[0, pl.ds(0, 128)])
    for j in range(ROWS_PER_TILE // 128):
        pltpu.sync_copy(rows_v.at[0, pl.ds(0, 128)],
                        accum_sh.at[pl.ds(rbase + j * 128, 128)])
    pltpu.sync_copy(zdeg_hbm, deg_v)
    pltpu.sync_copy(deg_v, deg_sh.at[pl.ds(rbase, ROWS_PER_TILE)])
    pltpu.sync_copy(ones_hbm, ones_v)

    table_hbm = hd2_hbm.at[cid]
    for j in range(2):
        pltpu.sync_copy(table_hbm.at[pl.ds(rbase + j * 256, 256)],
                        rows_v.at[0])
        pltpu.sync_copy(rows_v.at[0],
                        table_sh.at[pl.ds(rbase + j * 256, 256)])
    pltpu.sync_copy(table_hbm.at[pl.ds(rbase + 512, 128)],
                    rows_v.at[1, pl.ds(0, 128)])
    pltpu.sync_copy(rows_v.at[1, pl.ds(0, 128)],
                    table_sh.at[pl.ds(rbase + 512, 128)])
    plsc.subcore_barrier()

    row_base = sid * ROWS2D

    def start_idx(g, b):
        e0 = (row_base + g * CB) * 128
        r0 = row_base + g * CB
        pltpu.async_copy(src1d_hbm.at[pl.ds(e0, C)], idxs_v.at[b], sem_i[b])
        pltpu.async_copy(dst2d_hbm.at[pl.ds(r0, CB)], idxd_v.at[b], sem_i[b])

    def wait_idx(g, b):
        e0 = (row_base + g * CB) * 128
        r0 = row_base + g * CB
        pltpu.make_async_copy(src1d_hbm.at[pl.ds(e0, C)], idxs_v.at[b],
                              sem_i[b]).wait()
        pltpu.make_async_copy(dst2d_hbm.at[pl.ds(r0, CB)], idxd_v.at[b],
                              sem_i[b]).wait()

    def start_gathers(g, b):
        @pl.when(g < H_CHUNKS)
        def _():
            pltpu.async_copy(table_hbm.at[idxs_v.at[b]], rows_v.at[b],
                             sem_g[b])

        @pl.when(g >= H_CHUNKS)
        def _():
            pltpu.async_copy(table_sh.at[idxs_v.at[b]], rows_v.at[b],
                             sem_g[b])

    def wait_gathers(g, b):
        pltpu.make_async_copy(table_sh.at[idxs_v.at[b]], rows_v.at[b],
                              sem_g[b]).wait()

    def deg_cond(g):
        return (g < NCHUNK // 2) == (cid == 0)

    def start_scatters(g, b):
        for j in range(CB):
            pltpu.async_copy(rows_v.at[b, pl.ds(j * 128, 128)],
                             accum_sh.at[idxd_v.at[b, j]],
                             sem_s[b], add=True)

        @pl.when(deg_cond(g))
        def _():
            for j in range(CB):
                pltpu.async_copy(ones_v, deg_sh.at[idxd_v.at[b, j]],
                                 sem_s[b], add=True)

    def wait_scatters(g, b):
        for j in range(CB):
            pltpu.make_async_copy(rows_v.at[b, pl.ds(j * 128, 128)],
                                  accum_sh.at[idxd_v.at[b, j]],
                                  sem_s[b]).wait()

        @pl.when(deg_cond(g))
        def _():
            for j in range(CB):
                pltpu.make_async_copy(ones_v, deg_sh.at[idxd_v.at[b, j]],
                                      sem_s[b]).wait()

    for b in (0, 1):
        start_idx(b, b)
    for b in (0, 1):
        wait_idx(b, b)
        start_gathers(b, b)

    @pl.loop(0, NCHUNK - 2, step=2)
    def _pair(i):
        for b in (0, 1):
            g = i + b
            wait_gathers(g, b)
            start_scatters(g, b)
        for b in (0, 1):
            g = i + b
            wait_scatters(g, b)
            start_idx(g + 2, b)
            wait_idx(g + 2, b)
            start_gathers(g + 2, b)

    for b in (0, 1):
        g = NCHUNK - 2 + b
        wait_gathers(g, b)
        start_scatters(g, b)
    for b in (0, 1):
        g = NCHUNK - 2 + b
        wait_scatters(g, b)

    plsc.subcore_barrier()

    @pl.when(cid == 0)
    def _():
        for j in range(5):
            pltpu.sync_copy(accum_sh.at[pl.ds(rbase + j * 128, 128)],
                            rows_v.at[0, pl.ds(0, 128)])
            pltpu.sync_copy(rows_v.at[0, pl.ds(0, 128)],
                            sumL_hbm.at[pl.ds(rbase + j * 128, 128)])
        pltpu.sync_copy(deg_sh.at[pl.ds(rbase, ROWS_PER_TILE)], deg_v)
        pltpu.sync_copy(deg_v, deg0_hbm.at[pl.ds(rbase, ROWS_PER_TILE)])

    @pl.when(cid == 1)
    def _():
        for j in range(5):
            pltpu.sync_copy(accum_sh.at[pl.ds(rbase + j * 128, 128)],
                            rows_v.at[0, pl.ds(0, 128)])
            pltpu.sync_copy(rows_v.at[0, pl.ds(0, 128)],
                            sumR_hbm.at[pl.ds(rbase + j * 128, 128)])
        pltpu.sync_copy(deg_sh.at[pl.ds(rbase, ROWS_PER_TILE)], deg_v)
        pltpu.sync_copy(deg_v, deg1_hbm.at[pl.ds(rbase, ROWS_PER_TILE)])


def _sc_scatter(hd2, src1d, dst2d, zrows, zdeg, ones128):
    mesh = plsc.VectorSubcoreMesh(core_axis_name="c", subcore_axis_name="s")
    f = functools.partial(
        pl.kernel,
        out_type=[
            jax.ShapeDtypeStruct((N_PAD, DH), jnp.float32),
            jax.ShapeDtypeStruct((N_PAD, DH), jnp.float32),
            jax.ShapeDtypeStruct((N_PAD,), jnp.float32),
            jax.ShapeDtypeStruct((N_PAD,), jnp.float32),
        ],
        mesh=mesh,
        scratch_types=[
            pltpu.VMEM((2, C), jnp.int32),
            pltpu.VMEM((2, CB, 128), jnp.int32),
            pltpu.VMEM((2, C, DH), jnp.float32),
            pltpu.VMEM((128,), jnp.float32),
            pltpu.VMEM((ROWS_PER_TILE,), jnp.float32),
            pltpu.VMEM_SHARED((N_PAD, DH), jnp.float32),
            pltpu.VMEM_SHARED((N_PAD, DH), jnp.float32),
            pltpu.VMEM_SHARED((N_PAD,), jnp.float32),
            pltpu.SemaphoreType.DMA,
            pltpu.SemaphoreType.DMA,
            pltpu.SemaphoreType.DMA,
            pltpu.SemaphoreType.DMA,
            pltpu.SemaphoreType.DMA,
            pltpu.SemaphoreType.DMA,
        ],
        compiler_params=pltpu.CompilerParams(use_tc_tiling_on_sc=False),
    )(_sc_body)
    return f(hd2, src1d, dst2d, zrows, zdeg, ones128)


def _combine_body(sL_ref, sR_ref, d0_ref, d1_ref, hd_ref, agg_ref, w_ref, b_ref,
                  out_ref):
    deg = jnp.maximum(d0_ref[...] + d1_ref[...], 1.0)
    inv = (1.0 / deg)[:, None]
    agg = agg_ref[...]
    hnL = agg[:, :DH] + sL_ref[...] * inv
    hnR = agg[:, DH:] + sR_ref[...] * inv
    acc = jnp.dot(hd_ref[...], w_ref[0:D, :], preferred_element_type=jnp.float32)
    acc = acc + jnp.dot(hnL, w_ref[D:D + DH, :], preferred_element_type=jnp.float32)
    acc = acc + jnp.dot(hnR, w_ref[D + DH:2 * D, :], preferred_element_type=jnp.float32)
    out_ref[...] = acc + b_ref[...][None, :]


def _combine(sL, sR, d0, d1, H_dst, agg, W, b):
    R = 1024
    grid = (N_PAD // R,)
    return pl.pallas_call(
        _combine_body,
        grid=grid,
        in_specs=[
            pl.BlockSpec((R, DH), lambda i: (i, 0)),
            pl.BlockSpec((R, DH), lambda i: (i, 0)),
            pl.BlockSpec((R,), lambda i: (i,)),
            pl.BlockSpec((R,), lambda i: (i,)),
            pl.BlockSpec((R, D), lambda i: (i, 0)),
            pl.BlockSpec((R, D), lambda i: (i, 0)),
            pl.BlockSpec((2 * D, OUT), lambda i: (0, 0)),
            pl.BlockSpec((OUT,), lambda i: (0,)),
        ],
        out_specs=pl.BlockSpec((R, OUT), lambda i: (i, 0)),
        out_shape=jax.ShapeDtypeStruct((N, OUT), jnp.float32),
    )(sL, sR, d0, d1, H_dst, agg, W, b)


def kernel(H_src, H_dst, HBar_src, agg_HBar_dst, edge_index, W, b):
    hd2 = _hdelta(H_src, HBar_src)

    src = edge_index[0]
    dst = edge_index[1]
    pad = E_PAD - E
    src_pad = jnp.concatenate([src, jnp.zeros((pad,), jnp.int32)])
    dst_pad = jnp.concatenate([dst, jnp.full((pad,), N, jnp.int32)])
    dst2d = dst_pad.reshape(E_PAD // 128, 128)

    zrows = jnp.zeros((128, DH), jnp.float32)
    zdeg = jnp.zeros((ROWS_PER_TILE,), jnp.float32)
    ones128 = jnp.ones((128,), jnp.float32)

    sL, sR, d0, d1 = _sc_scatter(hd2, src_pad, dst2d, zrows, zdeg, ones128)

    return _combine(sL, sR, d0, d1, H_dst, agg_HBar_dst, W, b)
